# baseline (speedup 1.0000x reference)
import functools

import jax
import jax.numpy as jnp
from jax import lax
from jax.experimental import pallas as pl
from jax.experimental.pallas import tpu as pltpu
from jax.experimental.pallas import tpu_sc as plsc

N_ROWS = 1600
EMBED = 32
BATCH = 1024
D = 8
NUM_CORES = 2
NUM_SUBCORES = 16
NUM_WORKERS = NUM_CORES * NUM_SUBCORES
SAMPLES_PER_WORKER = BATCH // NUM_WORKERS
CHUNK = 80
NUM_CHUNKS = N_ROWS // CHUNK
LANES = 16

_mesh = plsc.VectorSubcoreMesh(core_axis_name="c", subcore_axis_name="s")


@functools.partial(
    pl.kernel,
    out_type=jax.ShapeDtypeStruct((BATCH * N_ROWS, EMBED), jnp.float32),
    mesh=_mesh,
    scratch_types=[
        pltpu.VMEM((NUM_CHUNKS, CHUNK), jnp.int32),
        pltpu.VMEM((N_ROWS, EMBED), jnp.float32),
        pltpu.SemaphoreType.DMA,
    ],
    compiler_params=pltpu.CompilerParams(use_tc_tiling_on_sc=False),
)
def _permute_rows(table, perm, out, idx_v, rows_v, sem):
    wid = lax.axis_index("s") * NUM_CORES + lax.axis_index("c")
    base_sample = wid * SAMPLES_PER_WORKER

    pltpu.sync_copy(perm, idx_v)
    base = base_sample * N_ROWS
    for j in range(NUM_CHUNKS):
        for c in range(CHUNK // LANES):
            sl = (j, pl.ds(c * LANES, LANES))
            idx_v[sl] = idx_v[sl] + base

    def body(s, carry):
        b = base_sample + s
        copies = [
            pltpu.async_copy(
                table.at[idx_v.at[j]],
                rows_v.at[pl.ds(j * CHUNK, CHUNK)],
                sem,
            )
            for j in range(NUM_CHUNKS)
        ]
        for cp in copies:
            cp.wait()
        pltpu.sync_copy(rows_v, out.at[pl.ds(b * N_ROWS, N_ROWS)])
        for j in range(NUM_CHUNKS):
            for c in range(CHUNK // LANES):
                sl = (j, pl.ds(c * LANES, LANES))
                idx_v[sl] = idx_v[sl] + N_ROWS
        return carry

    lax.fori_loop(0, SAMPLES_PER_WORKER, body, 0)


def kernel(M, permutator):
    b = M.shape[0]
    flat = M.reshape(b * N_ROWS, EMBED)
    perm2d = permutator.reshape(NUM_CHUNKS, CHUNK)
    out = _permute_rows(flat, perm2d)
    return out.reshape(b, N_ROWS // D, D, EMBED)


# speedup vs baseline: 42.2750x; 42.2750x over previous
import functools

import jax
import jax.numpy as jnp
from jax import lax
from jax.experimental import pallas as pl
from jax.experimental.pallas import tpu as pltpu
from jax.experimental.pallas import tpu_sc as plsc

N_ROWS = 1600
EMBED = 32
BATCH = 1024
D = 8
NUM_CORES = 2
NUM_SUBCORES = 16
NUM_WORKERS = NUM_CORES * NUM_SUBCORES
PAIRS = N_ROWS // 2
PAIRS_PER_WORKER = PAIRS // NUM_WORKERS
SLICE_ROWS = EMBED // 16
NBUF = 4

_mesh = plsc.VectorSubcoreMesh(core_axis_name="c", subcore_axis_name="s")


@functools.partial(
    pl.kernel,
    out_type=jax.ShapeDtypeStruct((N_ROWS, EMBED, BATCH), jnp.float32),
    mesh=_mesh,
    scratch_types=[
        pltpu.VMEM((PAIRS, 2), jnp.int32),
        pltpu.VMEM((NBUF, 2, SLICE_ROWS, BATCH), jnp.float32),
        [pltpu.SemaphoreType.DMA] * NBUF,
        [pltpu.SemaphoreType.DMA] * NBUF,
    ],
    compiler_params=pltpu.CompilerParams(needs_layout_passes=False),
)
def _permute_slabs(table, perm, out, perm_v, bufs, gsems, ssems):
    wid = lax.axis_index("s") * NUM_CORES + lax.axis_index("c")
    pair_base = wid * PAIRS_PER_WORKER
    slab_base = pair_base * 2

    pltpu.sync_copy(perm, perm_v)

    n_steps = PAIRS_PER_WORKER * 16

    def start_gather(k, b):
        m = lax.div(k, 16)
        h = lax.rem(k, 16)
        return pltpu.async_copy(
            table.at[perm_v.at[pair_base + m], pl.ds(h * SLICE_ROWS, SLICE_ROWS)],
            bufs.at[b],
            gsems[b],
        )

    def start_scatter(k, b):
        m = lax.div(k, 16)
        h = lax.rem(k, 16)
        return pltpu.async_copy(
            bufs.at[b],
            out.at[pl.ds(slab_base + 2 * m, 2), pl.ds(h * SLICE_ROWS, SLICE_ROWS)],
            ssems[b],
        )

    dummy_src = table.at[pl.ds(0, 2), pl.ds(0, SLICE_ROWS)]
    dummy_dst = out.at[pl.ds(slab_base, 2), pl.ds(0, SLICE_ROWS)]

    def wait_gather(b):
        pltpu.make_async_copy(dummy_src, bufs.at[b], gsems[b]).wait()

    def wait_scatter(b):
        pltpu.make_async_copy(bufs.at[b], dummy_dst, ssems[b]).wait()

    for b in range(NBUF):
        start_gather(jnp.int32(b), b)

    def body(i, carry):
        for b in range(NBUF):
            k = i * NBUF + b

            @pl.when(i > 0)
            def _():
                wait_scatter(b)

            wait_gather(b)
            start_scatter(k, b)

            @pl.when(k + NBUF < n_steps)
            def _():
                start_gather(k + NBUF, b)

        return carry

    lax.fori_loop(0, n_steps // NBUF, body, 0)

    for b in range(NBUF):
        wait_scatter(b)


def kernel(M, permutator):
    table = jnp.transpose(M, (1, 2, 3, 0)).reshape(N_ROWS, EMBED, BATCH)
    out = _permute_slabs(table, permutator.reshape(PAIRS, 2))
    return jnp.transpose(
        out.reshape(N_ROWS // D, D, EMBED, BATCH), (3, 0, 1, 2)
    )


# speedup vs baseline: 45.1751x; 1.0686x over previous
import functools

import jax
import jax.numpy as jnp
from jax import lax
from jax.experimental import pallas as pl
from jax.experimental.pallas import tpu as pltpu
from jax.experimental.pallas import tpu_sc as plsc

N_ROWS = 1600
EMBED = 32
BATCH = 1024
D = 8
NUM_CORES = 2
NUM_SUBCORES = 16
NUM_WORKERS = NUM_CORES * NUM_SUBCORES
PAIRS = N_ROWS // 2
PAIRS_PER_WORKER = PAIRS // NUM_WORKERS
SLICE_ROWS = EMBED // 8
NBUF = 4

_mesh = plsc.VectorSubcoreMesh(core_axis_name="c", subcore_axis_name="s")


@functools.partial(
    pl.kernel,
    out_type=jax.ShapeDtypeStruct((N_ROWS, EMBED, BATCH), jnp.float32),
    mesh=_mesh,
    scratch_types=[
        pltpu.VMEM((PAIRS_PER_WORKER, 2), jnp.int32),
        pltpu.VMEM((NBUF, 2, SLICE_ROWS, BATCH), jnp.float32),
        [pltpu.SemaphoreType.DMA] * NBUF,
        [pltpu.SemaphoreType.DMA] * NBUF,
    ],
    compiler_params=pltpu.CompilerParams(needs_layout_passes=False),
)
def _permute_slabs(table, perm, out, perm_v, bufs, gsems, ssems):
    wid = lax.axis_index("s") * NUM_CORES + lax.axis_index("c")
    pair_base = wid * PAIRS_PER_WORKER
    slab_base = pair_base * 2

    pltpu.sync_copy(perm.at[wid], perm_v)

    n_steps = PAIRS_PER_WORKER * 8

    def start_gather(k, b):
        m = lax.div(k, 8)
        h = lax.rem(k, 8)
        return pltpu.async_copy(
            table.at[perm_v.at[m], pl.ds(h * SLICE_ROWS, SLICE_ROWS)],
            bufs.at[b],
            gsems[b],
        )

    def start_scatter(k, b):
        m = lax.div(k, 8)
        h = lax.rem(k, 8)
        return pltpu.async_copy(
            bufs.at[b],
            out.at[pl.ds(slab_base + 2 * m, 2), pl.ds(h * SLICE_ROWS, SLICE_ROWS)],
            ssems[b],
        )

    dummy_src = table.at[pl.ds(0, 2), pl.ds(0, SLICE_ROWS)]
    dummy_dst = out.at[pl.ds(slab_base, 2), pl.ds(0, SLICE_ROWS)]

    def wait_gather(b):
        pltpu.make_async_copy(dummy_src, bufs.at[b], gsems[b]).wait()

    def wait_scatter(b):
        pltpu.make_async_copy(bufs.at[b], dummy_dst, ssems[b]).wait()

    for b in range(NBUF):
        start_gather(jnp.int32(b), b)

    def body(i, carry):
        for b in range(NBUF):
            k = i * NBUF + b

            @pl.when(i > 0)
            def _():
                wait_scatter(b)

            wait_gather(b)
            start_scatter(k, b)

            @pl.when(k + NBUF < n_steps)
            def _():
                start_gather(k + NBUF, b)

        return carry

    lax.fori_loop(0, n_steps // NBUF, body, 0)

    for b in range(NBUF):
        wait_scatter(b)


def kernel(M, permutator):
    table = jnp.transpose(M, (1, 2, 3, 0)).reshape(N_ROWS, EMBED, BATCH)
    out = _permute_slabs(
        table, permutator.reshape(NUM_WORKERS, PAIRS_PER_WORKER, 2)
    )
    return jnp.transpose(
        out.reshape(N_ROWS // D, D, EMBED, BATCH), (3, 0, 1, 2)
    )
